# Initial kernel scaffold; baseline (speedup 1.0000x reference)
import jax
import jax.numpy as jnp
from jax.experimental import pallas as pl


def kernel(token_ids, weight):
    raise NotImplementedError("write your pallas kernel here")


# speedup vs baseline: 1.8089x; 1.8089x over previous
import functools

import jax
import jax.numpy as jnp
from jax import lax
from jax.experimental import pallas as pl
from jax.experimental.pallas import tpu as pltpu
from jax.experimental.pallas import tpu_sc as plsc

VOCAB_SIZE = 1000000
D = 128
BATCH = 4096
SEQ = 200
B_TOTAL = BATCH * SEQ
NC, NS = 2, 16
NW = NC * NS
PER_W = B_TOTAL // NW
CH = 128
NCH = PER_W // CH
NBUF = 4

_mesh = plsc.VectorSubcoreMesh(core_axis_name="c", subcore_axis_name="s")


@functools.partial(
    pl.kernel,
    out_type=jax.ShapeDtypeStruct((NW * NCH, CH, D), jnp.float32),
    mesh=_mesh,
    scratch_types=[
        pltpu.VMEM((NCH, CH), jnp.int32),
        pltpu.VMEM((NBUF, CH, D), jnp.float32),
    ] + [pltpu.SemaphoreType.DMA] * (2 * NBUF),
)
def _sc_gather(table_hbm, idx_hbm, out_hbm, idx_v, rows_v, *sems):
    gsem = sems[:NBUF]
    osem = sems[NBUF:]
    wid = lax.axis_index("s") * NC + lax.axis_index("c")
    pltpu.sync_copy(idx_hbm.at[wid], idx_v)

    @pl.loop(0, NCH, step=NBUF)
    def _(g):
        gd = [
            pltpu.async_copy(table_hbm.at[idx_v.at[g + b]], rows_v.at[b], gsem[b])
            for b in range(NBUF)
        ]
        od = []
        for b in range(NBUF):
            gd[b].wait()
            od.append(
                pltpu.async_copy(rows_v.at[b], out_hbm.at[wid * NCH + g + b], osem[b])
            )
        for b in range(NBUF):
            od[b].wait()


def kernel(token_ids, weight):
    idx = token_ids.reshape(NW, NCH, CH).astype(jnp.int32)
    out = _sc_gather(weight, idx)
    return out.reshape(BATCH, SEQ, D)


# speedup vs baseline: 1.8679x; 1.0326x over previous
import functools

import jax
import jax.numpy as jnp
from jax import lax
from jax.experimental import pallas as pl
from jax.experimental.pallas import tpu as pltpu
from jax.experimental.pallas import tpu_sc as plsc

VOCAB_SIZE = 1000000
D = 128
BATCH = 4096
SEQ = 200
B_TOTAL = BATCH * SEQ
NC, NS = 2, 16
NW = NC * NS
PER_W = B_TOTAL // NW
CH = 128
NCH = PER_W // CH
NBUF = 4

M = NBUF * ((NCH - NBUF) // NBUF)

_mesh = plsc.VectorSubcoreMesh(core_axis_name="c", subcore_axis_name="s")


@functools.partial(
    pl.kernel,
    out_type=jax.ShapeDtypeStruct((NW * NCH, CH, D), jnp.float32),
    mesh=_mesh,
    scratch_types=[
        pltpu.VMEM((NCH, CH), jnp.int32),
        pltpu.VMEM((NBUF, CH, D), jnp.float32),
    ] + [pltpu.SemaphoreType.DMA] * (2 * NBUF),
)
def _sc_gather(table_hbm, idx_hbm, out_hbm, idx_v, rows_v, *sems):
    gsem = sems[:NBUF]
    osem = sems[NBUF:]
    wid = lax.axis_index("s") * NC + lax.axis_index("c")
    pltpu.sync_copy(idx_hbm.at[wid], idx_v)

    def g_desc(j, slot):
        return pltpu.make_async_copy(
            table_hbm.at[idx_v.at[j]], rows_v.at[slot], gsem[slot])

    def o_desc(j, slot):
        return pltpu.make_async_copy(
            rows_v.at[slot], out_hbm.at[wid * NCH + j], osem[slot])

    for b in range(NBUF):
        g_desc(b, b).start()
    g_desc(0, 0).wait()
    o_desc(0, 0).start()

    @pl.loop(1, M + 1, step=NBUF)
    def _(g):
        for b in range(NBUF):
            j = g + b
            s_prev = b
            s_cur = (b + 1) % NBUF
            o_desc(j - 1, s_prev).wait()
            g_desc(j - 1 + NBUF, s_prev).start()
            g_desc(j, s_cur).wait()
            o_desc(j, s_cur).start()

    for j in range(M + 1, NCH):
        o_desc(j - 1, (j - 1) % NBUF).wait()
        if j - 1 + NBUF < NCH:
            g_desc(j - 1 + NBUF, (j - 1) % NBUF).start()
        g_desc(j, j % NBUF).wait()
        o_desc(j, j % NBUF).start()
    o_desc(NCH - 1, (NCH - 1) % NBUF).wait()


def kernel(token_ids, weight):
    idx = token_ids.reshape(NW, NCH, CH).astype(jnp.int32)
    out = _sc_gather(weight, idx)
    return out.reshape(BATCH, SEQ, D)
